# Initial kernel scaffold; baseline (speedup 1.0000x reference)
import jax
import jax.numpy as jnp
from jax.experimental import pallas as pl


def kernel(h_idx, r_idx, t_idx, entity_emb, relation_emb):
    raise NotImplementedError("write your pallas kernel here")


# speedup vs baseline: 1.8557x; 1.8557x over previous
import functools

import jax
import jax.numpy as jnp
from jax import lax
from jax.experimental import pallas as pl
from jax.experimental.pallas import tpu as pltpu
from jax.experimental.pallas import tpu_sc as plsc

L = 16


@functools.cache
def _build(B, D, n_ent, n_rel):
    info = plsc.get_sparse_core_info()
    NC, NS = info.num_cores, info.num_subcores
    NW = NC * NS
    assert B % (8 * NW) == 0 and D % L == 0
    rows_w = B // NW
    CH = 128
    nch = rows_w // CH
    mesh = plsc.VectorSubcoreMesh(core_axis_name="c", subcore_axis_name="s")

    @functools.partial(
        pl.kernel,
        mesh=mesh,
        out_type=jax.ShapeDtypeStruct((B,), jnp.float32),
        scratch_types=[
            pltpu.VMEM((CH,), jnp.int32),
            pltpu.VMEM((CH,), jnp.int32),
            pltpu.VMEM((CH,), jnp.int32),
            pltpu.VMEM((CH,), jnp.int32),
            pltpu.VMEM((CH,), jnp.int32),
            pltpu.VMEM((CH,), jnp.int32),
            pltpu.VMEM((CH, D), jnp.float32),
            pltpu.VMEM((CH, D), jnp.float32),
            pltpu.VMEM((CH, D), jnp.float32),
            pltpu.VMEM((CH, D), jnp.float32),
            pltpu.VMEM((CH, D), jnp.float32),
            pltpu.VMEM((CH, D), jnp.float32),
            pltpu.VMEM((rows_w,), jnp.float32),
            pltpu.SemaphoreType.DMA,
            pltpu.SemaphoreType.DMA,
        ],
    )
    def scorer(h_hbm, r_hbm, t_hbm, ent_hbm, rel_hbm, out_hbm,
               hi0, hi1, ri0, ri1, ti0, ti1,
               eh0, eh1, er0, er1, et0, et1,
               outv, sem0, sem1):
        wid = lax.axis_index("s") * NC + lax.axis_index("c")
        base = wid * rows_w

        hidx, ridx, tidx = [hi0, hi1], [ri0, ri1], [ti0, ti1]
        eh, er, et = [eh0, eh1], [er0, er1], [et0, et1]
        sem = [sem0, sem1]

        def fetch(k, b):
            off = base + k * CH
            pltpu.sync_copy(h_hbm.at[pl.ds(off, CH)], hidx[b])
            pltpu.sync_copy(r_hbm.at[pl.ds(off, CH)], ridx[b])
            pltpu.sync_copy(t_hbm.at[pl.ds(off, CH)], tidx[b])
            d1 = pltpu.async_copy(ent_hbm.at[hidx[b]], eh[b], sem[b])
            d2 = pltpu.async_copy(rel_hbm.at[ridx[b]], er[b], sem[b])
            d3 = pltpu.async_copy(ent_hbm.at[tidx[b]], et[b], sem[b])
            return d1, d2, d3

        def compute(k, b):
            ehb, erb, etb = eh[b], er[b], et[b]
            lanes = lax.iota(jnp.int32, L)
            masks = [(lanes & (1 << s)) == 0 for s in range(4)]
            perms = [lanes ^ (1 << s) for s in range(4)]
            dnums = lax.GatherDimensionNumbers(
                offset_dims=(), collapsed_slice_dims=(0,), start_index_map=(0,))

            def permute(v, perm):
                return lax.gather(
                    v, perm[:, None], dnums, slice_sizes=(1,),
                    mode=lax.GatherScatterMode.PROMISE_IN_BOUNDS)

            def gbody(g, carry):
                rowbase = g * L
                ps = []
                for ri in range(L):
                    r = rowbase + ri
                    p = None
                    for c in range(D // L):
                        sl = pl.ds(c * L, L)
                        v = jnp.abs(ehb[r, sl] + erb[r, sl] - etb[r, sl])
                        p = v if p is None else p + v
                    ps.append(p)
                for s in range(4):
                    nxt = []
                    for j in range(0, len(ps), 2):
                        u, w = ps[j], ps[j + 1]
                        a = jnp.where(masks[s], u, w)
                        bsel = jnp.where(masks[s], w, u)
                        nxt.append(a + permute(bsel, perms[s]))
                    ps = nxt
                outv[pl.ds(k * CH + rowbase, L)] = -ps[0]
                return carry

            lax.fori_loop(0, CH // L, gbody, 0)

        descs = fetch(0, 0)
        for k in range(nch):
            b = k % 2
            nxt = fetch(k + 1, 1 - b) if k + 1 < nch else None
            for d in descs:
                d.wait()
            compute(k, b)
            descs = nxt

        pltpu.sync_copy(outv, out_hbm.at[pl.ds(base, rows_w)])

    return scorer


def kernel(h_idx, r_idx, t_idx, entity_emb, relation_emb):
    B, = h_idx.shape
    n_ent, D = entity_emb.shape
    n_rel, _ = relation_emb.shape
    scorer = _build(B, D, n_ent, n_rel)
    return scorer(h_idx.astype(jnp.int32), r_idx.astype(jnp.int32),
                  t_idx.astype(jnp.int32), entity_emb, relation_emb)
